# baseline (speedup 1.0000x reference)
import functools

import jax
import jax.numpy as jnp
from jax import lax
from jax.experimental import pallas as pl
from jax.experimental.pallas import tpu as pltpu
from jax.experimental.pallas import tpu_sc as plsc

N_TOK = 4096
D = 1024
H = 2048
C = 1024
E = 8
BT = 128
PTOT = 2 * N_TOK + E * BT
NBLK = PTOT // BT
CB = 512
NC, NS = 2, 16
NW = NC * NS
TPW = N_TOK // NW
SUBT = 16


def _router_kernel(x_ref, wg_ref, bg_ref,
                   p0_ref, p1_ref, w0_ref, w1_ref, be_ref):
    x = x_ref[...]
    logits = jax.lax.dot_general(
        x, wg_ref[...], (((1,), (0,)), ((), ())),
        preferred_element_type=jnp.float32) + bg_ref[...]
    lane = lax.broadcasted_iota(jnp.int32, (N_TOK, E), 1)
    m1 = jnp.max(logits, axis=1, keepdims=True)
    i1 = jnp.min(jnp.where(logits == m1, lane, E), axis=1, keepdims=True)
    masked = jnp.where(lane == i1, -jnp.inf, logits)
    m2 = jnp.max(masked, axis=1, keepdims=True)
    i2 = jnp.min(jnp.where(masked == m2, lane, E), axis=1, keepdims=True)
    e1 = jnp.exp(m2 - m1)
    den = 1.0 + e1
    w0_ref[...] = 1.0 / den
    w1_ref[...] = e1 / den

    ind = jnp.logical_or(lane == i1, lane == i2).astype(jnp.bfloat16)
    rio = lax.broadcasted_iota(jnp.int32, (CB, CB), 0)
    cio = lax.broadcasted_iota(jnp.int32, (CB, CB), 1)
    tri = (rio > cio).astype(jnp.bfloat16)
    carry = jnp.zeros((1, E), jnp.float32)
    excs = []
    for b in range(N_TOK // CB):
        blk = ind[b * CB:(b + 1) * CB, :]
        excs.append(jnp.dot(tri, blk, preferred_element_type=jnp.float32)
                    + carry)
        carry = carry + jnp.sum(blk.astype(jnp.float32), axis=0,
                                keepdims=True)
    exc = jnp.concatenate(excs, axis=0)
    tot = carry
    padded = jnp.ceil(tot * (1.0 / BT)) * BT
    r8 = lax.broadcasted_iota(jnp.int32, (E, E), 0)
    c8 = lax.broadcasted_iota(jnp.int32, (E, E), 1)
    upper = (r8 < c8).astype(jnp.bfloat16)
    off = jnp.dot(padded.astype(jnp.bfloat16), upper,
                  preferred_element_type=jnp.float32)
    ends = off + padded

    pos0 = jnp.zeros((N_TOK, 1), jnp.float32)
    pos1 = jnp.zeros((N_TOK, 1), jnp.float32)
    for e in range(E):
        slot_e = off[:, e:e + 1] + exc[:, e:e + 1]
        pos0 = pos0 + jnp.where(i1 == e, slot_e, 0.0)
        pos1 = pos1 + jnp.where(i2 == e, slot_e, 0.0)
    p0_ref[...] = pos0.astype(jnp.int32)
    p1_ref[...] = pos1.astype(jnp.int32)

    g_iota = (lax.broadcasted_iota(jnp.int32, (1, NBLK), 1)
              .astype(jnp.float32) * float(BT))
    bexp = jnp.zeros((1, NBLK), jnp.float32)
    for e in range(E):
        bexp = bexp + (g_iota >= ends[:, e:e + 1]).astype(jnp.float32)
    be_ref[...] = jnp.minimum(bexp, float(E - 1)).astype(jnp.int32)


def _ffn_kernel(be_ref, xs_ref, w1_ref, b1_ref, w2_ref, b2_ref, ws_ref,
                ys_ref):
    h = jnp.dot(xs_ref[...].astype(jnp.float32), w1_ref[0],
                preferred_element_type=jnp.float32)
    h = jnp.maximum(h + b1_ref[0], 0.0)
    o = jnp.dot(h, w2_ref[0], preferred_element_type=jnp.float32)
    ys_ref[...] = (o + b2_ref[0]) * ws_ref[...]


def _sc_scatter_impl(x_hbm, p0_hbm, p1_hbm, w0_hbm, w1_hbm, xs_hbm, ws_hbm,
                     xbuf, pbuf, wbuf, semx, sem0, sem1, sem2, sem3):
    wid = lax.axis_index("s") * NC + lax.axis_index("c")
    n0 = wid * TPW
    cx = pltpu.async_copy(x_hbm.at[pl.ds(n0, TPW)], xbuf, semx)
    pltpu.sync_copy(p0_hbm.at[pl.ds(n0, TPW)], pbuf.at[0])
    pltpu.sync_copy(p1_hbm.at[pl.ds(n0, TPW)], pbuf.at[1])
    pltpu.sync_copy(w0_hbm.at[pl.ds(n0, TPW)], wbuf.at[0])
    pltpu.sync_copy(w1_hbm.at[pl.ds(n0, TPW)], wbuf.at[1])
    cx.wait()
    c0 = pltpu.async_copy(xbuf, xs_hbm.at[pbuf.at[0]], sem0)
    c1 = pltpu.async_copy(xbuf, xs_hbm.at[pbuf.at[1]], sem1)
    c2 = pltpu.async_copy(wbuf.at[0], ws_hbm.at[pbuf.at[0]], sem2)
    c3 = pltpu.async_copy(wbuf.at[1], ws_hbm.at[pbuf.at[1]], sem3)
    c0.wait()
    c1.wait()
    c2.wait()
    c3.wait()


def _sc_combine_impl(ys_hbm, p0_hbm, p1_hbm, out_hbm,
                     i0a, i1a, i0b, i1b, r0a, r1a, r0b, r1b,
                     semA0, semB0, semA1, semB1, semW0, semW1):
    wid = lax.axis_index("s") * NC + lax.axis_index("c")
    n0 = wid * TPW
    nsub = TPW // SUBT
    bufs = [(i0a, i1a, r0a, r1a, semA0, semB0, semW0),
            (i0b, i1b, r0b, r1b, semA1, semB1, semW1)]

    def issue(t, s):
        i0, i1, r0, r1, semA, semB, _ = s
        base = n0 + t * SUBT
        pltpu.sync_copy(p0_hbm.at[pl.ds(base, SUBT)], i0)
        pltpu.sync_copy(p1_hbm.at[pl.ds(base, SUBT)], i1)
        return (pltpu.async_copy(ys_hbm.at[i0], r0, semA),
                pltpu.async_copy(ys_hbm.at[i1], r1, semB))

    gath = [None, None]
    pend = [None, None]
    gath[0] = issue(0, bufs[0])
    for t in range(nsub):
        p = t & 1
        if t + 1 < nsub:
            if pend[1 - p] is not None:
                pend[1 - p].wait()
            gath[1 - p] = issue(t + 1, bufs[1 - p])
        cA, cB = gath[p]
        cA.wait()
        cB.wait()
        _, _, r0, r1, _, _, semW = bufs[p]

        def row(i, _, r0=r0, r1=r1):
            for cc in range(C // 16):
                sl = pl.ds(cc * 16, 16)
                r0[i, sl] = r0[i, sl] + r1[i, sl]
            return 0

        lax.fori_loop(0, SUBT, row, 0)
        pend[p] = pltpu.async_copy(
            r0, out_hbm.at[pl.ds(n0 + t * SUBT, SUBT)], semW)
    for p in range(2):
        if pend[p] is not None:
            pend[p].wait()


@functools.cache
def _get_sc_kernels():
    mesh = plsc.VectorSubcoreMesh(
        core_axis_name="c", subcore_axis_name="s",
        num_cores=NC, num_subcores=NS)
    sc_scatter = pl.kernel(
        _sc_scatter_impl,
        out_type=[
            jax.ShapeDtypeStruct((PTOT, D // 2), jnp.int32),
            jax.ShapeDtypeStruct((PTOT,), jnp.float32),
        ],
        mesh=mesh,
        scratch_types=[
            pltpu.VMEM((TPW, D // 2), jnp.int32),
            pltpu.VMEM((2, TPW), jnp.int32),
            pltpu.VMEM((2, TPW), jnp.float32),
            pltpu.SemaphoreType.DMA,
            pltpu.SemaphoreType.DMA,
            pltpu.SemaphoreType.DMA,
            pltpu.SemaphoreType.DMA,
            pltpu.SemaphoreType.DMA,
        ],
    )
    sc_combine = pl.kernel(
        _sc_combine_impl,
        out_type=jax.ShapeDtypeStruct((N_TOK, C), jnp.float32),
        mesh=mesh,
        scratch_types=(
            [pltpu.VMEM((SUBT,), jnp.int32)] * 4
            + [pltpu.VMEM((SUBT, C), jnp.float32)] * 4
            + [pltpu.SemaphoreType.DMA] * 6
        ),
    )
    return sc_scatter, sc_combine


def _router(x, Wg, bg):
    return pl.pallas_call(
        _router_kernel,
        out_shape=[
            jax.ShapeDtypeStruct((N_TOK, 1), jnp.int32),
            jax.ShapeDtypeStruct((N_TOK, 1), jnp.int32),
            jax.ShapeDtypeStruct((N_TOK, 1), jnp.float32),
            jax.ShapeDtypeStruct((N_TOK, 1), jnp.float32),
            jax.ShapeDtypeStruct((1, NBLK), jnp.int32),
        ],
    )(x, Wg, bg.reshape(1, E))


def _ffn(be, xs, W1, b1, W2, b2, ws):
    grid_spec = pltpu.PrefetchScalarGridSpec(
        num_scalar_prefetch=1,
        grid=(NBLK,),
        in_specs=[
            pl.BlockSpec((BT, D), lambda g, be: (g, 0)),
            pl.BlockSpec((1, D, H), lambda g, be: (be[g], 0, 0)),
            pl.BlockSpec((1, 1, H), lambda g, be: (be[g], 0, 0)),
            pl.BlockSpec((1, H, C), lambda g, be: (be[g], 0, 0)),
            pl.BlockSpec((1, 1, C), lambda g, be: (be[g], 0, 0)),
            pl.BlockSpec((BT, 1), lambda g, be: (g, 0)),
        ],
        out_specs=pl.BlockSpec((BT, C), lambda g, be: (g, 0)),
    )
    return pl.pallas_call(
        _ffn_kernel,
        grid_spec=grid_spec,
        out_shape=jax.ShapeDtypeStruct((PTOT, C), jnp.float32),
    )(be, xs, W1, b1.reshape(E, 1, H), W2, b2.reshape(E, 1, C), ws)


def kernel(x, Wg, bg, W1, b1, W2, b2):
    sc_scatter, sc_combine = _get_sc_kernels()
    p0, p1, w0, w1, be = _router(x, Wg, bg)
    p0f = p0.reshape(N_TOK)
    p1f = p1.reshape(N_TOK)
    xpacked = jax.lax.bitcast_convert_type(
        x.astype(jnp.bfloat16).reshape(N_TOK, D // 2, 2), jnp.int32)
    xs_p, ws = sc_scatter(xpacked, p0f, p1f,
                          w0.reshape(N_TOK), w1.reshape(N_TOK))
    xs = jax.lax.bitcast_convert_type(xs_p, jnp.bfloat16).reshape(PTOT, D)
    ys = _ffn(be.reshape(NBLK), xs, W1, b1, W2, b2, ws.reshape(PTOT, 1))
    return sc_combine(ys, p0f, p1f)
